# Initial kernel scaffold; baseline (speedup 1.0000x reference)
import jax
import jax.numpy as jnp
from jax.experimental import pallas as pl


def kernel(x_feats, coords, W):
    raise NotImplementedError("write your pallas kernel here")


# speedup vs baseline: 42.0047x; 42.0047x over previous
import jax
import jax.numpy as jnp
from jax import lax
from jax.experimental import pallas as pl
from jax.experimental.pallas import tpu as pltpu
from jax.experimental.pallas import tpu_sc as plsc

N = 200000
COUT = 16
TBL = 1 << 24
PAD = 4096
ZPAD = 65536
NP = 204800
NC = 2
NS = 16
L = 16
NW = NC * NS

CBA = 2560
NCHA = (NP // NS) // CBA
ZB = 32768
ZCOPIES = (TBL // NW) // ZB

CB = 640
NCHB = (NP // NW) // CB
NCHUNKS = NP // CB

OFFS = [(dx, dy, dz) for dx in (-1, 0, 1) for dy in (-1, 0, 1) for dz in (-1, 0, 1)]


def _build_body(addr_hbm, feat_hbm, tbl_hbm, a_v, f_v, fidx_v, zero_v, sem):
    c = lax.axis_index("c")
    s = lax.axis_index("s")
    wid = c * NS + s

    def zfill(i, _):
        zero_v[pl.ds(i * L, L)] = jnp.zeros((L,), jnp.float32)
        return _
    lax.fori_loop(0, ZB // L, zfill, None)

    base = c * (TBL // NC) + s * (TBL // NW)
    copies = []
    for j in range(ZCOPIES):
        copies.append(
            pltpu.async_copy(zero_v, tbl_hbm.at[pl.ds(base + j * ZB, ZB)], sem))
    copies.append(pltpu.async_copy(
        zero_v.at[pl.ds(0, ZPAD // NW)],
        tbl_hbm.at[pl.ds(TBL + PAD + wid * (ZPAD // NW), ZPAD // NW)], sem))
    for cp in copies:
        cp.wait()
    plsc.subcore_barrier()

    rowbase = s * (NP // NS)
    for t in range(NCHA):
        rb = rowbase + t * CBA
        pltpu.sync_copy(addr_hbm.at[pl.ds(rb, CBA)], a_v)
        pltpu.sync_copy(feat_hbm.at[pl.ds(rb, CBA)], f_v)

        def gbody(g, _):
            a = a_v[pl.ds(g * L, L)]
            mine = (a >> 23) == c
            fa = jnp.where(mine, a, TBL + (a & (PAD - 1)))
            fidx_v[pl.ds(g * L, L)] = fa
            return _
        lax.fori_loop(0, CBA // L, gbody, None)
        pltpu.async_copy(f_v, tbl_hbm.at[fidx_v], sem).wait()


def _gather_body(tbl_hbm, uniq_hbm, f_hbm, u_v, idx_v, val_v, sem, gsem):
    c = lax.axis_index("c")
    s = lax.axis_index("s")
    wid = s * NC + c
    rowbase = wid * (NP // NW)

    for t in range(NCHB):
        rb = rowbase + t * CB
        pltpu.sync_copy(uniq_hbm.at[pl.ds(rb, CB)], u_v)

        def bbody(g, _):
            u = u_v[pl.ds(g * L, L)]
            ox = u >> 16
            oy = (u >> 8) & 255
            oz = u & 255
            mv = u < TBL
            bx = ox >= 1
            by = oy >= 1
            bz = oz >= 1
            q2 = u + u
            rg = lax.iota(jnp.int32, L) + (rb + g * L)
            zb = TBL + PAD + (rg & 2047) * 27
            for k, (dx, dy, dz) in enumerate(OFFS):
                ck = dx * 65536 + dy * 256 + dz
                mk = mv
                if dx < 0:
                    mk = mk & bx
                if dy < 0:
                    mk = mk & by
                if dz < 0:
                    mk = mk & bz
                q = (q2 + ck) & (TBL - 1)
                idx_v[pl.ds(k * CB + g * L, L)] = jnp.where(mk, q, zb + k)
            return _
        lax.fori_loop(0, CB // L, bbody, None)

        gathers = []
        for k in range(27):
            gathers.append(pltpu.async_copy(
                tbl_hbm.at[idx_v.at[pl.ds(k * CB, CB)]],
                val_v.at[pl.ds(k * CB, CB)], gsem))
        for cp in gathers:
            cp.wait()

        m = rb // CB
        pltpu.sync_copy(val_v, f_hbm.at[pl.ds(m * 27 * CB, 27 * CB)])


def _mm_body(f_ref, w_ref, o_ref):
    o_ref[...] = lax.dot_general(
        f_ref[0], w_ref[...], (((0,), (0,)), ((), ())),
        preferred_element_type=jnp.float32,
        precision=lax.Precision.HIGHEST)


@jax.jit
def kernel(x_feats, coords, W):
    x = coords[:, 1]
    y = coords[:, 2]
    z = coords[:, 3]
    lin = (x << 16) | (y << 8) | z
    olin = ((x >> 1) << 16) | ((y >> 1) << 8) | (z >> 1)

    lin_s, feat_s = lax.sort([lin, x_feats[:, 0]], num_keys=1)
    keep = jnp.concatenate([jnp.ones((1,), bool), lin_s[1:] != lin_s[:-1]])
    addr = jnp.where(keep, lin_s, TBL + (lin_s & (PAD - 1)))
    addr_p = jnp.concatenate(
        [addr, TBL + (jnp.arange(NP - N, dtype=jnp.int32) & (PAD - 1))])
    feat_p = jnp.concatenate([feat_s, jnp.zeros((NP - N,), jnp.float32)])

    uniq = jnp.unique(olin, size=N, fill_value=TBL)
    uniq_p = jnp.concatenate(
        [uniq.astype(jnp.int32), jnp.full((NP - N,), TBL, jnp.int32)])

    mesh = plsc.VectorSubcoreMesh(core_axis_name="c", subcore_axis_name="s")

    build = pl.kernel(
        _build_body,
        out_type=jax.ShapeDtypeStruct((TBL + PAD + ZPAD,), jnp.float32),
        mesh=mesh,
        scratch_types=[
            pltpu.VMEM((CBA,), jnp.int32),
            pltpu.VMEM((CBA,), jnp.float32),
            pltpu.VMEM((CBA,), jnp.int32),
            pltpu.VMEM((ZB,), jnp.float32),
            pltpu.SemaphoreType.DMA,
        ],
    )
    tbl = build(addr_p, feat_p)

    gather = pl.kernel(
        _gather_body,
        out_type=jax.ShapeDtypeStruct((NCHUNKS * 27 * CB,), jnp.float32),
        mesh=mesh,
        scratch_types=[
            pltpu.VMEM((CB,), jnp.int32),
            pltpu.VMEM((27 * CB,), jnp.int32),
            pltpu.VMEM((27 * CB,), jnp.float32),
            pltpu.SemaphoreType.DMA,
            pltpu.SemaphoreType.DMA,
        ],
    )
    f = gather(tbl, uniq_p).reshape(NCHUNKS, 27, CB)

    out = pl.pallas_call(
        _mm_body,
        out_shape=jax.ShapeDtypeStruct((NP, COUT), jnp.float32),
        grid=(NCHUNKS,),
        in_specs=[
            pl.BlockSpec((1, 27, CB), lambda m: (m, 0, 0)),
            pl.BlockSpec((27, COUT), lambda m: (0, 0)),
        ],
        out_specs=pl.BlockSpec((CB, COUT), lambda m: (m, 0)),
    )(f, W[:, 0, :])
    return out[:N]


# speedup vs baseline: 66.8788x; 1.5922x over previous
import jax
import jax.numpy as jnp
from jax import lax
from jax.experimental import pallas as pl
from jax.experimental.pallas import tpu as pltpu
from jax.experimental.pallas import tpu_sc as plsc
from jax._src.pallas import mpmd as _mpmd

N = 200000
COUT = 16
TBL = 1 << 24
PAD = 4096
ZPAD = 65536
NP = 204800
NC = 2
NS = 16
L = 16
NW = NC * NS

CBA = 2560
NCHA = (NP // NS) // CBA
ZB = 32768
ZCOPIES = (TBL // NW) // ZB

CB = 640
NCHB = (NP // NW) // CB
NCHUNKS = NP // CB

OFFS = [(dx, dy, dz) for dx in (-1, 0, 1) for dy in (-1, 0, 1) for dz in (-1, 0, 1)]


def _build_body(tblz_hbm, addr_hbm, feat_hbm, tbl_hbm, a_v, f_v, sem):
    del tblz_hbm
    c = lax.axis_index("c")
    s = lax.axis_index("s")
    wid = s * NC + c
    rb = wid * (NP // NW)
    pltpu.sync_copy(addr_hbm.at[pl.ds(rb, NP // NW)], a_v)
    pltpu.sync_copy(feat_hbm.at[pl.ds(rb, NP // NW)], f_v)
    pltpu.async_copy(f_v, tbl_hbm.at[a_v], sem).wait()


def _gather_body(tbl_hbm, uniq_hbm, f_hbm, u_v, idx_v, val_v, sem, gsem):
    c = lax.axis_index("c")
    s = lax.axis_index("s")
    wid = s * NC + c
    rowbase = wid * (NP // NW)

    for t in range(NCHB):
        rb = rowbase + t * CB
        pltpu.sync_copy(uniq_hbm.at[pl.ds(rb, CB)], u_v)

        def bbody(g, _):
            u = u_v[pl.ds(g * L, L)]
            ox = u >> 16
            oy = (u >> 8) & 255
            oz = u & 255
            mv = u < TBL
            bx = ox >= 1
            by = oy >= 1
            bz = oz >= 1
            q2 = u + u
            rg = lax.iota(jnp.int32, L) + (rb + g * L)
            zb = TBL + PAD + (rg & 2047) * 27
            for k, (dx, dy, dz) in enumerate(OFFS):
                ck = dx * 65536 + dy * 256 + dz
                mk = mv
                if dx < 0:
                    mk = mk & bx
                if dy < 0:
                    mk = mk & by
                if dz < 0:
                    mk = mk & bz
                q = (q2 + ck) & (TBL - 1)
                idx_v[pl.ds(k * CB + g * L, L)] = jnp.where(mk, q, zb + k)
            return _
        lax.fori_loop(0, CB // L, bbody, None)

        gathers = []
        for k in range(27):
            gathers.append(pltpu.async_copy(
                tbl_hbm.at[idx_v.at[pl.ds(k * CB, CB)]],
                val_v.at[pl.ds(k * CB, CB)], gsem))
        for cp in gathers:
            cp.wait()

        m = rb // CB
        pltpu.sync_copy(val_v, f_hbm.at[pl.ds(m * 27 * CB, 27 * CB)])


def _mm_body(f_ref, w_ref, o_ref):
    o_ref[...] = lax.dot_general(
        f_ref[0], w_ref[...], (((0,), (0,)), ((), ())),
        preferred_element_type=jnp.float32,
        precision=lax.Precision.HIGHEST)


@jax.jit
def kernel(x_feats, coords, W):
    x = coords[:, 1]
    y = coords[:, 2]
    z = coords[:, 3]
    lin = (x << 16) | (y << 8) | z
    olin = ((x >> 1) << 16) | ((y >> 1) << 8) | (z >> 1)

    lin_s, feat_s = lax.sort([lin, x_feats[:, 0]], num_keys=1)
    keep = jnp.concatenate([jnp.ones((1,), bool), lin_s[1:] != lin_s[:-1]])
    addr = jnp.where(keep, lin_s, TBL + (lin_s & (PAD - 1)))
    addr_p = jnp.concatenate(
        [addr, TBL + (jnp.arange(NP - N, dtype=jnp.int32) & (PAD - 1))])
    feat_p = jnp.concatenate([feat_s, jnp.zeros((NP - N,), jnp.float32)])

    uniq = jnp.unique(olin, size=N, fill_value=TBL)
    uniq_p = jnp.concatenate(
        [uniq.astype(jnp.int32), jnp.full((NP - N,), TBL, jnp.int32)])

    mesh = plsc.VectorSubcoreMesh(core_axis_name="c", subcore_axis_name="s")

    tblz = jnp.zeros((TBL + PAD + ZPAD,), jnp.float32)
    build = _mpmd._mpmd_map(
        [(mesh, _build_body)],
        out_types=jax.ShapeDtypeStruct((TBL + PAD + ZPAD,), jnp.float32),
        input_output_aliases={0: 0},
        scratch_types=[
            pltpu.VMEM((NP // NW,), jnp.int32),
            pltpu.VMEM((NP // NW,), jnp.float32),
            pltpu.SemaphoreType.DMA,
        ],
    )
    tbl = build(tblz, addr_p, feat_p)

    gather = pl.kernel(
        _gather_body,
        out_type=jax.ShapeDtypeStruct((NCHUNKS * 27 * CB,), jnp.float32),
        mesh=mesh,
        scratch_types=[
            pltpu.VMEM((CB,), jnp.int32),
            pltpu.VMEM((27 * CB,), jnp.int32),
            pltpu.VMEM((27 * CB,), jnp.float32),
            pltpu.SemaphoreType.DMA,
            pltpu.SemaphoreType.DMA,
        ],
    )
    f = gather(tbl, uniq_p).reshape(NCHUNKS, 27, CB)

    out = pl.pallas_call(
        _mm_body,
        out_shape=jax.ShapeDtypeStruct((NP, COUT), jnp.float32),
        grid=(NCHUNKS,),
        in_specs=[
            pl.BlockSpec((1, 27, CB), lambda m: (m, 0, 0)),
            pl.BlockSpec((27, COUT), lambda m: (0, 0)),
        ],
        out_specs=pl.BlockSpec((CB, COUT), lambda m: (m, 0)),
    )(f, W[:, 0, :])
    return out[:N]


# speedup vs baseline: 66.9326x; 1.0008x over previous
import jax
import jax.numpy as jnp
from jax import lax
from jax.experimental import pallas as pl
from jax.experimental.pallas import tpu as pltpu
from jax.experimental.pallas import tpu_sc as plsc
from jax._src.pallas import mpmd as _mpmd

N = 200000
COUT = 16
TBL = 1 << 24
PAD = 4096
ZPAD = 65536
NP = 204800
NC = 2
NS = 16
L = 16
NW = NC * NS

CBA = 2560
NCHA = (NP // NS) // CBA
ZB = 32768
ZCOPIES = (TBL // NW) // ZB

CB = 640
NCHB = (NP // NW) // CB
NCHUNKS = NP // CB

OFFS = [(dx, dy, dz) for dx in (-1, 0, 1) for dy in (-1, 0, 1) for dz in (-1, 0, 1)]


NSTR = 4
SCH = NP // NW // NSTR


def _build_body(tblz_hbm, addr_hbm, feat_hbm, tbl_hbm, *rest):
    del tblz_hbm
    a_vs = rest[:NSTR]
    f_vs = rest[NSTR:2 * NSTR]
    sem = rest[2 * NSTR]
    c = lax.axis_index("c")
    s = lax.axis_index("s")
    wid = s * NC + c
    rb = wid * (NP // NW)
    copies = []
    for i in range(NSTR):
        pltpu.sync_copy(addr_hbm.at[pl.ds(rb + i * SCH, SCH)], a_vs[i])
        pltpu.sync_copy(feat_hbm.at[pl.ds(rb + i * SCH, SCH)], f_vs[i])
        copies.append(pltpu.async_copy(f_vs[i], tbl_hbm.at[a_vs[i]], sem))
    for cp in copies:
        cp.wait()


def _gather_body(tbl_hbm, uniq_hbm, f_hbm, u_v, idx_v, val_v, sem, gsem):
    c = lax.axis_index("c")
    s = lax.axis_index("s")
    wid = s * NC + c
    rowbase = wid * (NP // NW)

    for t in range(NCHB):
        rb = rowbase + t * CB
        pltpu.sync_copy(uniq_hbm.at[pl.ds(rb, CB)], u_v)

        def bbody(g, _):
            u = u_v[pl.ds(g * L, L)]
            ox = u >> 16
            oy = (u >> 8) & 255
            oz = u & 255
            mv = u < TBL
            bx = ox >= 1
            by = oy >= 1
            bz = oz >= 1
            q2 = u + u
            rg = lax.iota(jnp.int32, L) + (rb + g * L)
            zb = TBL + PAD + (rg & 2047) * 27
            for k, (dx, dy, dz) in enumerate(OFFS):
                ck = dx * 65536 + dy * 256 + dz
                mk = mv
                if dx < 0:
                    mk = mk & bx
                if dy < 0:
                    mk = mk & by
                if dz < 0:
                    mk = mk & bz
                q = (q2 + ck) & (TBL - 1)
                idx_v[pl.ds(k * CB + g * L, L)] = jnp.where(mk, q, zb + k)
            return _
        lax.fori_loop(0, CB // L, bbody, None)

        gathers = []
        for k in range(27):
            gathers.append(pltpu.async_copy(
                tbl_hbm.at[idx_v.at[pl.ds(k * CB, CB)]],
                val_v.at[pl.ds(k * CB, CB)], gsem))
        for cp in gathers:
            cp.wait()

        m = rb // CB
        pltpu.sync_copy(val_v, f_hbm.at[pl.ds(m * 27 * CB, 27 * CB)])


def _mm_body(f_ref, w_ref, o_ref):
    o_ref[...] = lax.dot_general(
        f_ref[0], w_ref[...], (((0,), (0,)), ((), ())),
        preferred_element_type=jnp.float32,
        precision=lax.Precision.HIGHEST)


@jax.jit
def kernel(x_feats, coords, W):
    x = coords[:, 1]
    y = coords[:, 2]
    z = coords[:, 3]
    lin = (x << 16) | (y << 8) | z
    olin = ((x >> 1) << 16) | ((y >> 1) << 8) | (z >> 1)

    lin_s, feat_s = lax.sort([lin, x_feats[:, 0]], num_keys=1)
    keep = jnp.concatenate([jnp.ones((1,), bool), lin_s[1:] != lin_s[:-1]])
    addr = jnp.where(keep, lin_s, TBL + (lin_s & (PAD - 1)))
    addr_p = jnp.concatenate(
        [addr, TBL + (jnp.arange(NP - N, dtype=jnp.int32) & (PAD - 1))])
    feat_p = jnp.concatenate([feat_s, jnp.zeros((NP - N,), jnp.float32)])

    uniq = jnp.unique(olin, size=N, fill_value=TBL)
    uniq_p = jnp.concatenate(
        [uniq.astype(jnp.int32), jnp.full((NP - N,), TBL, jnp.int32)])

    mesh = plsc.VectorSubcoreMesh(core_axis_name="c", subcore_axis_name="s")

    tblz = jnp.zeros((TBL + PAD + ZPAD,), jnp.float32)
    build = _mpmd._mpmd_map(
        [(mesh, _build_body)],
        out_types=jax.ShapeDtypeStruct((TBL + PAD + ZPAD,), jnp.float32),
        input_output_aliases={0: 0},
        scratch_types=(
            [pltpu.VMEM((SCH,), jnp.int32) for _ in range(NSTR)]
            + [pltpu.VMEM((SCH,), jnp.float32) for _ in range(NSTR)]
            + [pltpu.SemaphoreType.DMA]
        ),
    )
    tbl = build(tblz, addr_p, feat_p)

    gather = pl.kernel(
        _gather_body,
        out_type=jax.ShapeDtypeStruct((NCHUNKS * 27 * CB,), jnp.float32),
        mesh=mesh,
        scratch_types=[
            pltpu.VMEM((CB,), jnp.int32),
            pltpu.VMEM((27 * CB,), jnp.int32),
            pltpu.VMEM((27 * CB,), jnp.float32),
            pltpu.SemaphoreType.DMA,
            pltpu.SemaphoreType.DMA,
        ],
    )
    f = gather(tbl, uniq_p).reshape(NCHUNKS, 27, CB)

    out = pl.pallas_call(
        _mm_body,
        out_shape=jax.ShapeDtypeStruct((NP, COUT), jnp.float32),
        grid=(NCHUNKS,),
        in_specs=[
            pl.BlockSpec((1, 27, CB), lambda m: (m, 0, 0)),
            pl.BlockSpec((27, COUT), lambda m: (0, 0)),
        ],
        out_specs=pl.BlockSpec((CB, COUT), lambda m: (m, 0)),
    )(f, W[:, 0, :])
    return out[:N]


# speedup vs baseline: 68.9829x; 1.0306x over previous
import jax
import jax.numpy as jnp
from jax import lax
from jax.experimental import pallas as pl
from jax.experimental.pallas import tpu as pltpu
from jax.experimental.pallas import tpu_sc as plsc
from jax._src.pallas import mpmd as _mpmd

N = 200000
COUT = 16
TBL = 1 << 24
PAD = 4096
ZPAD = 65536
NP = 204800
NC = 2
NS = 16
L = 16
NW = NC * NS

CBA = 2560
NCHA = (NP // NS) // CBA
ZB = 32768
ZCOPIES = (TBL // NW) // ZB

CB = 640
NCHB = (NP // NW) // CB
NCHUNKS = NP // CB

OFFS = [(dx, dy, dz) for dx in (-1, 0, 1) for dy in (-1, 0, 1) for dz in (-1, 0, 1)]


NSTR = 4
SCH = NP // NW // NSTR


def _build_body(tblz_hbm, addr_hbm, feat_hbm, tbl_hbm, *rest):
    del tblz_hbm
    a_vs = rest[:NSTR]
    f_vs = rest[NSTR:2 * NSTR]
    sem = rest[2 * NSTR]
    c = lax.axis_index("c")
    s = lax.axis_index("s")
    wid = s * NC + c
    rb = wid * (NP // NW)
    copies = []
    for i in range(NSTR):
        pltpu.sync_copy(addr_hbm.at[pl.ds(rb + i * SCH, SCH)], a_vs[i])
        pltpu.sync_copy(feat_hbm.at[pl.ds(rb + i * SCH, SCH)], f_vs[i])
        copies.append(pltpu.async_copy(f_vs[i], tbl_hbm.at[a_vs[i]], sem))
    for cp in copies:
        cp.wait()


def _gather_body(tbl_hbm, uniq_hbm, f_hbm, u_v, idx_v, val_v, sem, gsem):
    c = lax.axis_index("c")
    s = lax.axis_index("s")
    wid = s * NC + c
    rowbase = wid * (NP // NW)

    for t in range(NCHB):
        rb = rowbase + t * CB
        pltpu.sync_copy(uniq_hbm.at[pl.ds(rb, CB)], u_v)

        def bbody(g, _):
            u = u_v[pl.ds(g * L, L)]
            ox = u >> 16
            oy = (u >> 8) & 255
            oz = u & 255
            mv = u < TBL
            bx = ox >= 1
            by = oy >= 1
            bz = oz >= 1
            q2 = u + u
            rg = lax.iota(jnp.int32, L) + (rb + g * L)
            zb = TBL + PAD + (rg & 2047) * 27
            for k, (dx, dy, dz) in enumerate(OFFS):
                ck = dx * 65536 + dy * 256 + dz
                mk = mv
                if dx < 0:
                    mk = mk & bx
                if dy < 0:
                    mk = mk & by
                if dz < 0:
                    mk = mk & bz
                q = (q2 + ck) & (TBL - 1)
                idx_v[pl.ds(k * CB + g * L, L)] = jnp.where(mk, q, zb + k)
            return _
        lax.fori_loop(0, CB // L, bbody, None)

        gathers = []
        for k in range(27):
            gathers.append(pltpu.async_copy(
                tbl_hbm.at[idx_v.at[pl.ds(k * CB, CB)]],
                val_v.at[pl.ds(k * CB, CB)], gsem))
        for cp in gathers:
            cp.wait()

        m = rb // CB
        pltpu.sync_copy(val_v, f_hbm.at[pl.ds(m * 27 * CB, 27 * CB)])


def _mm_body(f_ref, w_ref, o_ref):
    o_ref[...] = lax.dot_general(
        f_ref[0], w_ref[...], (((0,), (0,)), ((), ())),
        preferred_element_type=jnp.float32,
        precision=lax.Precision.HIGHEST)


@jax.jit
def kernel(x_feats, coords, W):
    x = coords[:, 1]
    y = coords[:, 2]
    z = coords[:, 3]
    lin = (x << 16) | (y << 8) | z
    olin = ((x >> 1) << 16) | ((y >> 1) << 8) | (z >> 1)

    lin_s, feat_s = lax.sort([lin, x_feats[:, 0]], num_keys=1)
    keep = jnp.concatenate([jnp.ones((1,), bool), lin_s[1:] != lin_s[:-1]])
    addr = jnp.where(keep, lin_s, TBL + (lin_s & (PAD - 1)))
    addr_p = jnp.concatenate(
        [addr, TBL + (jnp.arange(NP - N, dtype=jnp.int32) & (PAD - 1))])
    feat_p = jnp.concatenate([feat_s, jnp.zeros((NP - N,), jnp.float32)])

    uniq = jnp.unique(olin, size=N, fill_value=TBL)
    uniq_p = jnp.concatenate(
        [uniq.astype(jnp.int32), jnp.full((NP - N,), TBL, jnp.int32)])

    mesh = plsc.VectorSubcoreMesh(core_axis_name="c", subcore_axis_name="s")

    tblz = jnp.zeros((TBL + PAD + ZPAD,), jnp.float32)
    build = _mpmd._mpmd_map(
        [(mesh, _build_body)],
        out_types=jax.ShapeDtypeStruct((TBL + PAD + ZPAD,), jnp.float32),
        input_output_aliases={0: 0},
        scratch_types=(
            [pltpu.VMEM((SCH,), jnp.int32) for _ in range(NSTR)]
            + [pltpu.VMEM((SCH,), jnp.float32) for _ in range(NSTR)]
            + [pltpu.SemaphoreType.DMA]
        ),
    )
    tbl = build(tblz, addr_p, feat_p)

    gather = pl.kernel(
        _gather_body,
        out_type=jax.ShapeDtypeStruct((NCHUNKS * 27 * CB,), jnp.float32),
        mesh=mesh,
        scratch_types=[
            pltpu.VMEM((CB,), jnp.int32),
            pltpu.VMEM((27 * CB,), jnp.int32),
            pltpu.VMEM((27 * CB,), jnp.float32),
            pltpu.SemaphoreType.DMA,
            pltpu.SemaphoreType.DMA,
        ],
    )
    f = gather(tbl, uniq_p).reshape(NCHUNKS, 27, CB)

    out = pl.pallas_call(
        _mm_body,
        out_shape=jax.ShapeDtypeStruct((N, COUT), jnp.float32),
        grid=(N + CB - 1) // CB,
        in_specs=[
            pl.BlockSpec((1, 27, CB), lambda m: (m, 0, 0)),
            pl.BlockSpec((27, COUT), lambda m: (0, 0)),
        ],
        out_specs=pl.BlockSpec((CB, COUT), lambda m: (m, 0)),
    )(f, W[:, 0, :])
    return out
